# Initial kernel scaffold; baseline (speedup 1.0000x reference)
import jax
import jax.numpy as jnp
from jax.experimental import pallas as pl


def kernel(x, pos, batch, W1, b1, W2, b2, Wfc, bfc):
    raise NotImplementedError("write your pallas kernel here")


# speedup vs baseline: 3.8330x; 3.8330x over previous
import jax
import jax.numpy as jnp
from jax import lax
from jax.experimental import pallas as pl
from jax.experimental.pallas import tpu as pltpu

F = 128
NUM_SEG = 1000
SEG_PAD = 1024
BLK = 400


def _body(batch_smem, x_ref, bcol_ref, W1_ref, b1_ref, w2_ref, b2_ref,
          Wfc_ref, bfc_ref, out_ref,
          acc_sum, acc_max, acc_n, acc_cnt, acc_d, acc_m):
    i = pl.program_id(0)
    nblk = pl.num_programs(0)

    @pl.when(i == 0)
    def _init():
        acc_sum[...] = jnp.zeros_like(acc_sum)
        acc_max[...] = jnp.full_like(acc_max, -jnp.inf)
        acc_n[...] = jnp.zeros_like(acc_n)
        acc_cnt[...] = jnp.zeros_like(acc_cnt)
        acc_d[...] = jnp.zeros_like(acc_d)
        acc_m[...] = jnp.full_like(acc_m, -jnp.inf)

    x = x_ref[...]
    h = jnp.dot(x, W1_ref[...], preferred_element_type=jnp.float32)
    h = h + b1_ref[...]
    h = jnp.where(h > 0, h, 0.01 * h)
    s = jnp.sum(h * w2_ref[...], axis=1, keepdims=True) + b2_ref[...]
    bcol = bcol_ref[...]

    s0 = batch_smem[0, 0, 0]
    s1 = batch_smem[0, 0, BLK - 1]

    def seg_step(sid, carry):
        mask = bcol == sid
        xs = jnp.where(mask, x, 0.0)
        xm = jnp.where(mask, x, -jnp.inf)
        sum_blk = jnp.sum(xs, axis=0, keepdims=True)
        max_blk = jnp.max(xm, axis=0, keepdims=True)
        cnt_blk = jnp.sum(mask.astype(jnp.float32), axis=0, keepdims=True)
        bm = jnp.max(jnp.where(mask, s, -jnp.inf), axis=0, keepdims=True)
        m_old = acc_m[pl.ds(sid, 1), :]
        m_new = jnp.maximum(m_old, bm)
        scale = jnp.where(m_new == -jnp.inf, 0.0, jnp.exp(m_old - m_new))
        ex = jnp.where(mask, jnp.exp(s - m_new), 0.0)
        acc_sum[pl.ds(sid, 1), :] += sum_blk
        acc_max[pl.ds(sid, 1), :] = jnp.maximum(acc_max[pl.ds(sid, 1), :], max_blk)
        acc_cnt[pl.ds(sid, 1), :] += cnt_blk
        acc_m[pl.ds(sid, 1), :] = m_new
        acc_d[pl.ds(sid, 1), :] = acc_d[pl.ds(sid, 1), :] * scale \
            + jnp.sum(ex, axis=0, keepdims=True)
        acc_n[pl.ds(sid, 1), :] = acc_n[pl.ds(sid, 1), :] * scale \
            + jnp.sum(ex * x, axis=0, keepdims=True)
        return carry

    lax.fori_loop(s0, s1 + 1, seg_step, 0)

    @pl.when(i == nblk - 1)
    def _finish():
        sump = acc_sum[:NUM_SEG, :]
        maxp = acc_max[:NUM_SEG, :]
        maxp = jnp.where(maxp == -jnp.inf, 0.0, maxp)
        cnt = acc_cnt[:NUM_SEG, :]
        meanp = sump / jnp.maximum(cnt, 1.0)
        attn = acc_n[:NUM_SEG, :] / (acc_d[:NUM_SEG, :] + 1e-16)
        cat = jnp.concatenate([maxp, meanp, sump, attn], axis=1)
        out_ref[...] = jnp.dot(cat, Wfc_ref[...],
                               preferred_element_type=jnp.float32) + bfc_ref[...]


def kernel(x, pos, batch, W1, b1, W2, b2, Wfc, bfc):
    del pos
    n = x.shape[0]
    assert n % BLK == 0, n
    nblk = n // BLK
    batch = batch.astype(jnp.int32)
    batch3 = batch.reshape(nblk, 1, BLK)
    bcol = batch.reshape(n, 1)
    b1r = b1.reshape(1, F)
    w2r = W2.reshape(1, F)
    b2r = b2.reshape(1, 1)
    bfcr = bfc.reshape(1, F)

    out = pl.pallas_call(
        _body,
        grid=(nblk,),
        in_specs=[
            pl.BlockSpec((1, 1, BLK), lambda i: (i, 0, 0), memory_space=pltpu.SMEM),
            pl.BlockSpec((BLK, F), lambda i: (i, 0)),
            pl.BlockSpec((BLK, 1), lambda i: (i, 0)),
            pl.BlockSpec((F, F), lambda i: (0, 0)),
            pl.BlockSpec((1, F), lambda i: (0, 0)),
            pl.BlockSpec((1, F), lambda i: (0, 0)),
            pl.BlockSpec((1, 1), lambda i: (0, 0)),
            pl.BlockSpec((4 * F, F), lambda i: (0, 0)),
            pl.BlockSpec((1, F), lambda i: (0, 0)),
        ],
        out_specs=pl.BlockSpec((NUM_SEG, F), lambda i: (0, 0)),
        out_shape=jax.ShapeDtypeStruct((NUM_SEG, F), jnp.float32),
        scratch_shapes=[
            pltpu.VMEM((SEG_PAD, F), jnp.float32),
            pltpu.VMEM((SEG_PAD, F), jnp.float32),
            pltpu.VMEM((SEG_PAD, F), jnp.float32),
            pltpu.VMEM((SEG_PAD, 1), jnp.float32),
            pltpu.VMEM((SEG_PAD, 1), jnp.float32),
            pltpu.VMEM((SEG_PAD, 1), jnp.float32),
        ],
        compiler_params=pltpu.CompilerParams(
            dimension_semantics=("arbitrary",),
        ),
    )(batch3, x, bcol, W1, b1r, w2r, b2r, Wfc, bfcr)
    return out
